# baseline (device time: 103996 ns/iter reference)
import functools

import jax
import jax.numpy as jnp
from jax import lax
from jax.experimental import pallas as pl
from jax.experimental.pallas import tpu as pltpu

N_DEV = 8


def kernel(table, idx):
    rows_per, d = table.shape
    n = idx.shape[0]

    my = lax.axis_index("i")
    local = idx - my * rows_per
    owned = (local >= 0) & (local < rows_per)
    safe = jnp.where(owned, local, 0)
    part = jnp.where(owned[:, None], table[safe], 0.0).astype(jnp.bfloat16)

    def body(x_ref, out_ref, comm_ref, send_sems, recv_sems):
        my_pos = lax.axis_index("i")
        left = lax.rem(my_pos - 1 + N_DEV, N_DEV)
        right = lax.rem(my_pos + 1, N_DEV)

        barrier_sem = pltpu.get_barrier_semaphore()
        for nbr in (left, right):
            pl.semaphore_signal(
                barrier_sem, inc=1,
                device_id=(nbr,), device_id_type=pl.DeviceIdType.MESH,
            )
        pl.semaphore_wait(barrier_sem, 2)

        comm_ref[0] = x_ref[...]
        out_ref[...] = x_ref[...].astype(jnp.float32)

        for h in range(N_DEV - 1):
            rdma = pltpu.make_async_remote_copy(
                src_ref=comm_ref.at[h],
                dst_ref=comm_ref.at[h + 1],
                send_sem=send_sems.at[h],
                recv_sem=recv_sems.at[h],
                device_id=(right,),
                device_id_type=pl.DeviceIdType.MESH,
            )
            rdma.start()
            rdma.wait()
            out_ref[...] = out_ref[...] + comm_ref[h + 1].astype(jnp.float32)

        @functools.partial(
            pl.run_scoped, second_barrier=pltpu.SemaphoreType.REGULAR
        )
        def _(second_barrier):
            for nbr in (left, right):
                pl.semaphore_signal(
                    second_barrier, inc=1,
                    device_id=(nbr,), device_id_type=pl.DeviceIdType.MESH,
                )
            pl.semaphore_wait(second_barrier, 2)

    out = pl.pallas_call(
        body,
        out_shape=jax.ShapeDtypeStruct((n, d), jnp.float32),
        in_specs=[pl.BlockSpec(memory_space=pltpu.VMEM)],
        out_specs=pl.BlockSpec(memory_space=pltpu.VMEM),
        scratch_shapes=[
            pltpu.VMEM((N_DEV, n, d), jnp.bfloat16),
            pltpu.SemaphoreType.DMA((N_DEV - 1,)),
            pltpu.SemaphoreType.DMA((N_DEV - 1,)),
        ],
        compiler_params=pltpu.CompilerParams(collective_id=0),
    )(part)
    return out


# device time: 39591 ns/iter; 2.6268x vs baseline; 2.6268x over previous
import jax
import jax.numpy as jnp
from jax import lax
from jax.experimental import pallas as pl
from jax.experimental.pallas import tpu as pltpu

N_DEV = 8


def kernel(table, idx):
    rows_per, d = table.shape
    n = idx.shape[0]

    my = lax.axis_index("i")
    local = idx - my * rows_per
    owned = (local >= 0) & (local < rows_per)
    safe = jnp.where(owned, local, 0)
    part = jnp.where(owned[:, None], table[safe], 0.0).astype(jnp.bfloat16)

    h1, h2, h3 = n // 2, n // 4, n // 8

    def body(x_ref, out_ref, acc_ref, rbuf_ref, send_sems, recv_sems):
        p = lax.axis_index("i")
        q = lax.rem(p, 4)
        bz = p // 4
        by = q // 2
        bx = lax.bitwise_xor(lax.rem(q, 2), by)
        px = lax.bitwise_xor(p, 1)
        py = lax.bitwise_xor(p, 3)
        pz = lax.bitwise_xor(p, 4)

        barrier_sem = pltpu.get_barrier_semaphore()
        for nbr in (px, py, pz):
            pl.semaphore_signal(
                barrier_sem, inc=1,
                device_id=(nbr,), device_id_type=pl.DeviceIdType.MESH,
            )
        pl.semaphore_wait(barrier_sem, 3)

        acc_ref[...] = x_ref[...]

        off_x = bx * h1
        off_xy = off_x + by * h2
        off_xyz = off_xy + bz * h3

        rs_rounds = [
            (px, (1 - bx) * h1, off_x, h1, 0, 0),
            (py, off_x + (1 - by) * h2, off_xy, h2, h1, 1),
            (pz, off_xy + (1 - bz) * h3, off_xyz, h3, h1 + h2, 2),
        ]
        for partner, send_off, my_off, size, roff, s in rs_rounds:
            rdma = pltpu.make_async_remote_copy(
                src_ref=acc_ref.at[pl.ds(send_off, size)],
                dst_ref=rbuf_ref.at[pl.ds(roff, size)],
                send_sem=send_sems.at[s],
                recv_sem=recv_sems.at[s],
                device_id=(partner,),
                device_id_type=pl.DeviceIdType.MESH,
            )
            rdma.start()
            rdma.wait()
            acc_ref[pl.ds(my_off, size)] = (
                acc_ref[pl.ds(my_off, size)] + rbuf_ref[pl.ds(roff, size)]
            )

        ag_rounds = [
            (pz, off_xyz, h3, 3),
            (py, off_xy, h2, 4),
            (px, off_x, h1, 5),
        ]
        for partner, send_off, size, s in ag_rounds:
            rdma = pltpu.make_async_remote_copy(
                src_ref=acc_ref.at[pl.ds(send_off, size)],
                dst_ref=acc_ref.at[pl.ds(send_off, size)],
                send_sem=send_sems.at[s],
                recv_sem=recv_sems.at[s],
                device_id=(partner,),
                device_id_type=pl.DeviceIdType.MESH,
            )
            rdma.start()
            rdma.wait()

        out_ref[...] = acc_ref[...].astype(jnp.float32)

    out = pl.pallas_call(
        body,
        out_shape=jax.ShapeDtypeStruct((n, d), jnp.float32),
        in_specs=[pl.BlockSpec(memory_space=pltpu.VMEM)],
        out_specs=pl.BlockSpec(memory_space=pltpu.VMEM),
        scratch_shapes=[
            pltpu.VMEM((n, d), jnp.bfloat16),
            pltpu.VMEM((h1 + h2 + h3, d), jnp.bfloat16),
            pltpu.SemaphoreType.DMA((6,)),
            pltpu.SemaphoreType.DMA((6,)),
        ],
        compiler_params=pltpu.CompilerParams(collective_id=0),
    )(part)
    return out


# device time: 28164 ns/iter; 3.6925x vs baseline; 1.4057x over previous
import jax
import jax.numpy as jnp
from jax import lax
from jax.experimental import pallas as pl
from jax.experimental.pallas import tpu as pltpu

N_DEV = 8


def kernel(table, idx):
    rows_per, d = table.shape
    n = idx.shape[0]

    my = lax.axis_index("i")
    local = idx - my * rows_per
    owned = (local >= 0) & (local < rows_per)
    safe = jnp.where(owned, local, 0)
    part = jnp.where(owned[:, None], table[safe], 0.0).astype(jnp.bfloat16)

    hc = n // N_DEV

    def chunk_off(q):
        qq = lax.rem(q, 4)
        qz = q // 4
        qy = qq // 2
        qx = lax.bitwise_xor(lax.rem(qq, 2), qy)
        return (qx * 4 + qy * 2 + qz) * hc

    def body(x_ref, out_ref, rbuf_ref, s1, r1, s2, r2):
        p = lax.axis_index("i")

        barrier_sem = pltpu.get_barrier_semaphore()
        for j in range(1, N_DEV):
            pl.semaphore_signal(
                barrier_sem, inc=1,
                device_id=(lax.bitwise_xor(p, j),),
                device_id_type=pl.DeviceIdType.MESH,
            )
        pl.semaphore_wait(barrier_sem, N_DEV - 1)

        sends1 = []
        for j in range(1, N_DEV):
            q = lax.bitwise_xor(p, j)
            rdma = pltpu.make_async_remote_copy(
                src_ref=x_ref.at[pl.ds(chunk_off(q), hc)],
                dst_ref=rbuf_ref.at[j - 1],
                send_sem=s1.at[j - 1],
                recv_sem=r1.at[j - 1],
                device_id=(q,),
                device_id_type=pl.DeviceIdType.MESH,
            )
            rdma.start()
            sends1.append(rdma)

        my_off = chunk_off(p)
        acc = x_ref[pl.ds(my_off, hc)]
        for j in range(1, N_DEV):
            sends1[j - 1].wait_recv()
            acc = acc + rbuf_ref[j - 1]
        out_ref[pl.ds(my_off, hc)] = acc

        sends2 = []
        for j in range(1, N_DEV):
            q = lax.bitwise_xor(p, j)
            rdma = pltpu.make_async_remote_copy(
                src_ref=out_ref.at[pl.ds(my_off, hc)],
                dst_ref=out_ref.at[pl.ds(my_off, hc)],
                send_sem=s2.at[j - 1],
                recv_sem=r2.at[j - 1],
                device_id=(q,),
                device_id_type=pl.DeviceIdType.MESH,
            )
            rdma.start()
            sends2.append(rdma)
        for rdma in sends2:
            rdma.wait_recv()
        for rdma in sends1:
            rdma.wait_send()
        for rdma in sends2:
            rdma.wait_send()

    out = pl.pallas_call(
        body,
        out_shape=jax.ShapeDtypeStruct((n, d), jnp.bfloat16),
        in_specs=[pl.BlockSpec(memory_space=pltpu.VMEM)],
        out_specs=pl.BlockSpec(memory_space=pltpu.VMEM),
        scratch_shapes=[
            pltpu.VMEM((N_DEV - 1, hc, d), jnp.bfloat16),
            pltpu.SemaphoreType.DMA((N_DEV - 1,)),
            pltpu.SemaphoreType.DMA((N_DEV - 1,)),
            pltpu.SemaphoreType.DMA((N_DEV - 1,)),
            pltpu.SemaphoreType.DMA((N_DEV - 1,)),
        ],
        compiler_params=pltpu.CompilerParams(collective_id=0),
    )(part)
    return out


# device time: 26222 ns/iter; 3.9660x vs baseline; 1.0741x over previous
import jax
import jax.numpy as jnp
from jax import lax
from jax.experimental import pallas as pl
from jax.experimental.pallas import tpu as pltpu

N_DEV = 8
_J_ORDER = (6, 7, 5, 2, 4, 3, 1)


def kernel(table, idx):
    rows_per, d = table.shape
    n = idx.shape[0]

    my = lax.axis_index("i")
    local = idx - my * rows_per
    owned = (local >= 0) & (local < rows_per)
    safe = jnp.where(owned, local, 0)
    gathered = table[safe].astype(jnp.bfloat16)
    maskf = owned.astype(jnp.bfloat16)

    hc = n // N_DEV

    def chunk_off(q):
        qq = lax.rem(q, 4)
        qz = q // 4
        qy = qq // 2
        qx = lax.bitwise_xor(lax.rem(qq, 2), qy)
        return (qx * 4 + qy * 2 + qz) * hc

    def body(g_ref, mask_ref, out_ref, part_ref, rbuf_ref, s1, r1, s2, r2):
        p = lax.axis_index("i")

        barrier_sem = pltpu.get_barrier_semaphore()
        for j in range(1, N_DEV):
            pl.semaphore_signal(
                barrier_sem, inc=1,
                device_id=(lax.bitwise_xor(p, j),),
                device_id_type=pl.DeviceIdType.MESH,
            )

        part_ref[...] = g_ref[...] * mask_ref[...][:, None]

        pl.semaphore_wait(barrier_sem, N_DEV - 1)

        sends1 = {}
        for j in _J_ORDER:
            q = lax.bitwise_xor(p, j)
            rdma = pltpu.make_async_remote_copy(
                src_ref=part_ref.at[pl.ds(chunk_off(q), hc)],
                dst_ref=rbuf_ref.at[j - 1],
                send_sem=s1.at[j - 1],
                recv_sem=r1.at[j - 1],
                device_id=(q,),
                device_id_type=pl.DeviceIdType.MESH,
            )
            rdma.start()
            sends1[j] = rdma

        my_off = chunk_off(p)
        acc = part_ref[pl.ds(my_off, hc)]
        for j in _J_ORDER:
            sends1[j].wait_recv()
            acc = acc + rbuf_ref[j - 1]
        out_ref[pl.ds(my_off, hc)] = acc

        sends2 = {}
        for j in _J_ORDER:
            q = lax.bitwise_xor(p, j)
            rdma = pltpu.make_async_remote_copy(
                src_ref=out_ref.at[pl.ds(my_off, hc)],
                dst_ref=out_ref.at[pl.ds(my_off, hc)],
                send_sem=s2.at[j - 1],
                recv_sem=r2.at[j - 1],
                device_id=(q,),
                device_id_type=pl.DeviceIdType.MESH,
            )
            rdma.start()
            sends2[j] = rdma
        for j in _J_ORDER:
            sends2[j].wait_recv()
        for j in _J_ORDER:
            sends1[j].wait_send()
            sends2[j].wait_send()

    out = pl.pallas_call(
        body,
        out_shape=jax.ShapeDtypeStruct((n, d), jnp.bfloat16),
        in_specs=[
            pl.BlockSpec(memory_space=pltpu.VMEM),
            pl.BlockSpec(memory_space=pltpu.VMEM),
        ],
        out_specs=pl.BlockSpec(memory_space=pltpu.VMEM),
        scratch_shapes=[
            pltpu.VMEM((n, d), jnp.bfloat16),
            pltpu.VMEM((N_DEV - 1, hc, d), jnp.bfloat16),
            pltpu.SemaphoreType.DMA((N_DEV - 1,)),
            pltpu.SemaphoreType.DMA((N_DEV - 1,)),
            pltpu.SemaphoreType.DMA((N_DEV - 1,)),
            pltpu.SemaphoreType.DMA((N_DEV - 1,)),
        ],
        compiler_params=pltpu.CompilerParams(collective_id=0),
    )(gathered, maskf)
    return out


# device time: 26128 ns/iter; 3.9803x vs baseline; 1.0036x over previous
import jax
import jax.numpy as jnp
from jax import lax
from jax.experimental import pallas as pl
from jax.experimental.pallas import tpu as pltpu

N_DEV = 8
C = 48
_J_ORDER = (6, 7, 5, 2, 4, 3, 1)


def _chunk_slot(q):
    qq = q % 4
    qz = q // 4
    qy = qq // 2
    qx = (qq % 2) ^ qy
    return qx * 4 + qy * 2 + qz


def kernel(table, idx):
    rows_per, d = table.shape
    n = idx.shape[0]
    hc = n // N_DEV

    my = lax.axis_index("i")
    t_my = _chunk_slot(my)

    idxc = idx.reshape(N_DEV, hc)
    local = idxc[None, :, :] - (jnp.arange(N_DEV) * rows_per)[:, None, None]
    owned = (local >= 0) & (local < rows_per)
    rank = jnp.cumsum(owned, axis=-1) - 1
    kar = jnp.arange(C)
    sel = owned[:, :, None, :] & (rank[:, :, None, :] == kar[None, None, :, None])
    pos = (sel * jnp.arange(hc)[None, None, None, :]).sum(-1).astype(jnp.int32)
    srcrow = (sel * jnp.where(owned, local, 0)[:, :, None, :]).sum(-1)
    count = owned.sum(-1)
    valid = kar[None, None, :] < count[:, :, None]

    rows_mine = srcrow[my].reshape(-1)
    valid_mine = valid[my].reshape(-1)
    b = (
        table[rows_mine].astype(jnp.bfloat16)
        * valid_mine[:, None].astype(jnp.bfloat16)
    )

    posm = pos[:, t_my, :]
    validm = valid[:, t_my, :].astype(jnp.bfloat16)

    def placement(posall, valall, s):
        m = lax.broadcasted_iota(jnp.int32, (N_DEV, C), 0) == s
        prow = jnp.sum(posall * m, axis=0, keepdims=True)
        vrow = jnp.sum(valall * m.astype(jnp.bfloat16), axis=0, keepdims=True)
        eq = lax.broadcasted_iota(jnp.int32, (hc, C), 0) == prow
        return eq.astype(jnp.bfloat16) * vrow

    def body(b_ref, pos_ref, val_ref, out_ref, rbuf_ref, s1, r1, s2, r2):
        p = lax.axis_index("i")

        barrier_sem = pltpu.get_barrier_semaphore()
        for j in range(1, N_DEV):
            pl.semaphore_signal(
                barrier_sem, inc=1,
                device_id=(lax.bitwise_xor(p, j),),
                device_id_type=pl.DeviceIdType.MESH,
            )
        pl.semaphore_wait(barrier_sem, N_DEV - 1)

        sends1 = {}
        for j in _J_ORDER:
            q = lax.bitwise_xor(p, j)
            rdma = pltpu.make_async_remote_copy(
                src_ref=b_ref.at[pl.ds(_chunk_slot(q) * C, C)],
                dst_ref=rbuf_ref.at[j - 1],
                send_sem=s1.at[j - 1],
                recv_sem=r1.at[j - 1],
                device_id=(q,),
                device_id_type=pl.DeviceIdType.MESH,
            )
            rdma.start()
            sends1[j] = rdma

        tp = _chunk_slot(p)
        posall = pos_ref[...]
        valall = val_ref[...]
        acc = lax.dot_general(
            placement(posall, valall, p),
            b_ref[pl.ds(tp * C, C)],
            (((1,), (0,)), ((), ())),
            preferred_element_type=jnp.float32,
        )
        for j in _J_ORDER:
            sends1[j].wait_recv()
            s = lax.bitwise_xor(p, j)
            acc = acc + lax.dot_general(
                placement(posall, valall, s),
                rbuf_ref[j - 1],
                (((1,), (0,)), ((), ())),
                preferred_element_type=jnp.float32,
            )
        my_off = tp * hc
        out_ref[pl.ds(my_off, hc)] = acc.astype(jnp.bfloat16)

        sends2 = {}
        for j in _J_ORDER:
            q = lax.bitwise_xor(p, j)
            rdma = pltpu.make_async_remote_copy(
                src_ref=out_ref.at[pl.ds(my_off, hc)],
                dst_ref=out_ref.at[pl.ds(my_off, hc)],
                send_sem=s2.at[j - 1],
                recv_sem=r2.at[j - 1],
                device_id=(q,),
                device_id_type=pl.DeviceIdType.MESH,
            )
            rdma.start()
            sends2[j] = rdma
        for j in _J_ORDER:
            sends2[j].wait_recv()
        for j in _J_ORDER:
            sends1[j].wait_send()
            sends2[j].wait_send()

    out = pl.pallas_call(
        body,
        out_shape=jax.ShapeDtypeStruct((n, d), jnp.bfloat16),
        in_specs=[
            pl.BlockSpec(memory_space=pltpu.VMEM),
            pl.BlockSpec(memory_space=pltpu.VMEM),
            pl.BlockSpec(memory_space=pltpu.VMEM),
        ],
        out_specs=pl.BlockSpec(memory_space=pltpu.VMEM),
        scratch_shapes=[
            pltpu.VMEM((N_DEV - 1, C, d), jnp.bfloat16),
            pltpu.SemaphoreType.DMA((N_DEV - 1,)),
            pltpu.SemaphoreType.DMA((N_DEV - 1,)),
            pltpu.SemaphoreType.DMA((N_DEV - 1,)),
            pltpu.SemaphoreType.DMA((N_DEV - 1,)),
        ],
        compiler_params=pltpu.CompilerParams(collective_id=0),
    )(b, posm, validm)
    return out


# device time: 23578 ns/iter; 4.4107x vs baseline; 1.1082x over previous
import jax
import jax.numpy as jnp
from jax import lax
from jax.experimental import pallas as pl
from jax.experimental.pallas import tpu as pltpu

N_DEV = 8
C = 48
_J_ORDER = (6, 7, 5, 2, 4, 3, 1)


def _chunk_slot(q):
    qq = q % 4
    qz = q // 4
    qy = qq // 2
    qx = (qq % 2) ^ qy
    return qx * 4 + qy * 2 + qz


def kernel(table, idx):
    rows_per, d = table.shape
    n = idx.shape[0]
    hc = n // N_DEV

    my = lax.axis_index("i")
    t_my = _chunk_slot(my)

    idxc = idx.reshape(N_DEV, hc)
    kar = jnp.arange(C)
    iar = jnp.arange(hc)

    local_me = idxc - my * rows_per
    owned_me = (local_me >= 0) & (local_me < rows_per)
    rank_me = jnp.cumsum(owned_me, axis=-1) - 1
    sel_me = owned_me[:, None, :] & (rank_me[:, None, :] == kar[None, :, None])
    pos_me = (sel_me * iar[None, None, :]).sum(-1)
    valid_me = kar[None, :] < owned_me.sum(-1)[:, None]
    srcrow = jnp.take_along_axis(
        jnp.where(owned_me, local_me, 0), pos_me, axis=-1
    )
    b = (
        table[srcrow.reshape(-1)].astype(jnp.bfloat16)
        * valid_me.reshape(-1)[:, None].astype(jnp.bfloat16)
    )

    chunk_idx = idxc[t_my]
    local_r = chunk_idx[None, :] - (jnp.arange(N_DEV) * rows_per)[:, None]
    owned_r = (local_r >= 0) & (local_r < rows_per)
    rank_r = jnp.cumsum(owned_r, axis=-1) - 1
    sel_r = owned_r[:, None, :] & (rank_r[:, None, :] == kar[None, :, None])
    posm = (sel_r * iar[None, None, :]).sum(-1).astype(jnp.int32)
    validm = (kar[None, :] < owned_r.sum(-1)[:, None]).astype(jnp.bfloat16)

    def placement(posall, valall, s):
        m = lax.broadcasted_iota(jnp.int32, (N_DEV, C), 0) == s
        prow = jnp.sum(posall * m, axis=0, keepdims=True)
        vrow = jnp.sum(valall * m.astype(jnp.bfloat16), axis=0, keepdims=True)
        eq = lax.broadcasted_iota(jnp.int32, (hc, C), 0) == prow
        return eq.astype(jnp.bfloat16) * vrow

    def body(b_ref, pos_ref, val_ref, out_ref, rbuf_ref, s1, r1, s2, r2):
        p = lax.axis_index("i")

        barrier_sem = pltpu.get_barrier_semaphore()
        for j in range(1, N_DEV):
            pl.semaphore_signal(
                barrier_sem, inc=1,
                device_id=(lax.bitwise_xor(p, j),),
                device_id_type=pl.DeviceIdType.MESH,
            )
        pl.semaphore_wait(barrier_sem, N_DEV - 1)

        sends1 = {}
        for j in _J_ORDER:
            q = lax.bitwise_xor(p, j)
            rdma = pltpu.make_async_remote_copy(
                src_ref=b_ref.at[pl.ds(_chunk_slot(q) * C, C)],
                dst_ref=rbuf_ref.at[j - 1],
                send_sem=s1.at[j - 1],
                recv_sem=r1.at[j - 1],
                device_id=(q,),
                device_id_type=pl.DeviceIdType.MESH,
            )
            rdma.start()
            sends1[j] = rdma

        tp = _chunk_slot(p)
        posall = pos_ref[...]
        valall = val_ref[...]
        acc = lax.dot_general(
            placement(posall, valall, p),
            b_ref[pl.ds(tp * C, C)],
            (((1,), (0,)), ((), ())),
            preferred_element_type=jnp.float32,
        )
        for j in _J_ORDER:
            sends1[j].wait_recv()
            s = lax.bitwise_xor(p, j)
            acc = acc + lax.dot_general(
                placement(posall, valall, s),
                rbuf_ref[j - 1],
                (((1,), (0,)), ((), ())),
                preferred_element_type=jnp.float32,
            )
        my_off = tp * hc
        out_ref[pl.ds(my_off, hc)] = acc.astype(jnp.bfloat16)

        sends2 = {}
        for j in _J_ORDER:
            q = lax.bitwise_xor(p, j)
            rdma = pltpu.make_async_remote_copy(
                src_ref=out_ref.at[pl.ds(my_off, hc)],
                dst_ref=out_ref.at[pl.ds(my_off, hc)],
                send_sem=s2.at[j - 1],
                recv_sem=r2.at[j - 1],
                device_id=(q,),
                device_id_type=pl.DeviceIdType.MESH,
            )
            rdma.start()
            sends2[j] = rdma
        for j in _J_ORDER:
            sends2[j].wait_recv()
        for j in _J_ORDER:
            sends1[j].wait_send()
            sends2[j].wait_send()

    out = pl.pallas_call(
        body,
        out_shape=jax.ShapeDtypeStruct((n, d), jnp.bfloat16),
        in_specs=[
            pl.BlockSpec(memory_space=pltpu.VMEM),
            pl.BlockSpec(memory_space=pltpu.VMEM),
            pl.BlockSpec(memory_space=pltpu.VMEM),
        ],
        out_specs=pl.BlockSpec(memory_space=pltpu.VMEM),
        scratch_shapes=[
            pltpu.VMEM((N_DEV - 1, C, d), jnp.bfloat16),
            pltpu.SemaphoreType.DMA((N_DEV - 1,)),
            pltpu.SemaphoreType.DMA((N_DEV - 1,)),
            pltpu.SemaphoreType.DMA((N_DEV - 1,)),
            pltpu.SemaphoreType.DMA((N_DEV - 1,)),
        ],
        compiler_params=pltpu.CompilerParams(collective_id=0),
    )(b, posm, validm)
    return out
